# baseline (speedup 1.0000x reference)
import functools

import jax
import jax.numpy as jnp
import numpy as np
from jax.experimental import pallas as pl
from jax.experimental.pallas import tpu as pltpu
from jax.experimental.pallas import tpu_sc as plsc

_NVEC = [1000000, 1000000, 1000000]
_NSC = 2
_ICHUNK = 2048
_K = 8
_BB = 2048


def _sc_gather_rows(U, flat_idx, n_rows, dim):
    half = n_rows // _NSC
    nch = half // _ICHUNK
    mesh = plsc.ScalarSubcoreMesh(axis_name="c", num_cores=_NSC)

    @functools.partial(
        pl.kernel,
        mesh=mesh,
        out_type=jax.ShapeDtypeStruct((n_rows, dim), jnp.float32),
        scratch_types=[
            pltpu.SMEM((_ICHUNK,), jnp.int32),
            pltpu.SMEM((_ICHUNK,), jnp.int32),
            pltpu.SemaphoreType.DMA,
            pltpu.SemaphoreType.DMA,
        ],
    )
    def gather_kernel(table_hbm, idx_hbm, out_hbm, idx_a, idx_b, sem_i,
                      sem_d):
        base = jax.lax.axis_index("c") * half

        def idx_copy(c, buf):
            return pltpu.make_async_copy(
                idx_hbm.at[pl.ds(base + c * _ICHUNK, _ICHUNK)], buf, sem_i
            )

        idx_copy(0, idx_a).start()

        def issue_rows(c, buf):
            off = base + c * _ICHUNK

            @pl.loop(0, _ICHUNK, step=_K)
            def _(j0):
                for jj in range(_K):
                    j = j0 + jj
                    pltpu.make_async_copy(
                        table_hbm.at[pl.ds(buf[j], 1)],
                        out_hbm.at[pl.ds(off + j, 1)],
                        sem_d,
                    ).start()

        @pl.loop(0, nch, step=2)
        def _(c):
            idx_copy(c, idx_a).wait()
            @pl.when(c + 1 < nch)
            def _():
                idx_copy(c + 1, idx_b).start()
            issue_rows(c, idx_a)

            @pl.when(c + 1 < nch)
            def _():
                idx_copy(c + 1, idx_b).wait()
                @pl.when(c + 2 < nch)
                def _():
                    idx_copy(c + 2, idx_a).start()
                issue_rows(c + 1, idx_b)

        pltpu.make_async_copy(
            table_hbm.at[pl.ds(0, half)], out_hbm.at[pl.ds(base, half)], sem_d
        ).wait()

    return gather_kernel(U, flat_idx)


def _mlp_body(nmod, g_ref, t_ref, w0_ref, wt_ref, b0_ref, w1_ref, b1_ref,
              o_ref):
    acc = None
    for k in range(nmod):
        d = jnp.dot(g_ref[k], w0_ref[k], preferred_element_type=jnp.float32)
        acc = d if acc is None else acc + d
    h = jnp.tanh(acc + t_ref[...] * wt_ref[...] + b0_ref[...])
    o_ref[...] = jnp.sum(h * w1_ref[...], axis=1, keepdims=True) + b1_ref[...]


def kernel(batch_ind, batch_t, U, W0, b0, W1, b1):
    B, nmod = batch_ind.shape
    dim = U.shape[1]
    nhid = W0.shape[0]

    offs = jnp.asarray(np.cumsum([0] + _NVEC[:-1]), dtype=jnp.int32)
    fi = batch_ind.astype(jnp.int32).T + offs[:, None]
    flat_idx = fi.reshape(-1)

    G = _sc_gather_rows(U, flat_idx, B * nmod, dim)
    G3 = G.reshape(nmod, B, dim)

    t2 = batch_t.reshape(B, 1)
    W0x = W0[:, : nmod * dim].T.reshape(nmod, dim, nhid)
    w0t = W0[:, nmod * dim:].T
    b0r = b0.reshape(1, nhid)
    b1r = b1.reshape(1, 1)

    out = pl.pallas_call(
        functools.partial(_mlp_body, nmod),
        grid=(B // _BB,),
        in_specs=[
            pl.BlockSpec((nmod, _BB, dim), lambda i: (0, i, 0)),
            pl.BlockSpec((_BB, 1), lambda i: (i, 0)),
            pl.BlockSpec((nmod, dim, nhid), lambda i: (0, 0, 0)),
            pl.BlockSpec((1, nhid), lambda i: (0, 0)),
            pl.BlockSpec((1, nhid), lambda i: (0, 0)),
            pl.BlockSpec((1, nhid), lambda i: (0, 0)),
            pl.BlockSpec((1, 1), lambda i: (0, 0)),
        ],
        out_specs=pl.BlockSpec((_BB, 1), lambda i: (i, 0)),
        out_shape=jax.ShapeDtypeStruct((B, 1), jnp.float32),
    )(G3, t2, W0x, w0t, b0r, W1, b1r)
    return out


# speedup vs baseline: 1.1790x; 1.1790x over previous
import functools

import jax
import jax.numpy as jnp
import numpy as np
from jax.experimental import pallas as pl
from jax.experimental.pallas import tpu as pltpu
from jax.experimental.pallas import tpu_sc as plsc

_NVEC = [1000000, 1000000, 1000000]
_NC, _NS = 2, 16
_NW = _NC * _NS
_GCHUNK = 768
_BB = 2048


def _sc_gather_groups(U4, q_idx, n_rows):
    b_per_w = n_rows // _NW
    nch = b_per_w // _GCHUNK
    mesh = plsc.VectorSubcoreMesh(core_axis_name="c", subcore_axis_name="s")

    @functools.partial(
        pl.kernel,
        mesh=mesh,
        out_type=jax.ShapeDtypeStruct((n_rows, 128), jnp.float32),
        scratch_types=[
            pltpu.VMEM((b_per_w,), jnp.int32),
            pltpu.VMEM((_GCHUNK, 128), jnp.float32),
            pltpu.SemaphoreType.DMA,
        ],
    )
    def gather_kernel(table_hbm, idx_hbm, out_hbm, idx_v, rows_v, sem):
        wid = jax.lax.axis_index("s") * _NC + jax.lax.axis_index("c")
        base = wid * b_per_w
        pltpu.sync_copy(idx_hbm.at[pl.ds(base, b_per_w)], idx_v)

        @pl.loop(0, nch)
        def _(c):
            off = c * _GCHUNK
            pltpu.async_copy(
                table_hbm.at[idx_v.at[pl.ds(off, _GCHUNK)]], rows_v, sem
            ).wait()
            pltpu.sync_copy(rows_v, out_hbm.at[pl.ds(base + off, _GCHUNK)])

    return gather_kernel(U4, q_idx)


def _mlp_body(nmod, dim, g_ref, sel_ref, t_ref, w0_ref, wt_ref, b0_ref,
              w1_ref, b1_ref, o_ref):
    bb = g_ref.shape[1]
    lane_grp = jax.lax.broadcasted_iota(jnp.int32, (bb, 128), 1) // dim
    acc = None
    for k in range(nmod):
        m = lane_grp == sel_ref[k]
        gk = jnp.where(m, g_ref[k], 0.0)
        d = jnp.dot(gk, w0_ref[k], preferred_element_type=jnp.float32)
        acc = d if acc is None else acc + d
    h = jnp.tanh(acc + t_ref[...] * wt_ref[...] + b0_ref[...])
    o_ref[...] = jnp.sum(h * w1_ref[...], axis=1, keepdims=True) + b1_ref[...]


def kernel(batch_ind, batch_t, U, W0, b0, W1, b1):
    B, nmod = batch_ind.shape
    dim = U.shape[1]
    nhid = W0.shape[0]
    gpr = 128 // dim

    offs = jnp.asarray(np.cumsum([0] + _NVEC[:-1]), dtype=jnp.int32)
    fi = batch_ind.astype(jnp.int32).T + offs[:, None]
    q = (fi // gpr).reshape(-1)
    sel3 = (fi % gpr).reshape(nmod, B, 1)

    U4 = U.reshape(U.shape[0] * dim // 128, 128)
    G4 = _sc_gather_groups(U4, q, B * nmod)
    G43 = G4.reshape(nmod, B, 128)

    t2 = batch_t.reshape(B, 1)
    W0x = W0[:, : nmod * dim].T
    w0big = jnp.tile(W0x.reshape(nmod, 1, dim, nhid), (1, gpr, 1, 1))
    w0big = w0big.reshape(nmod, 128, nhid)
    w0t = W0[:, nmod * dim:].T
    b0r = b0.reshape(1, nhid)
    b1r = b1.reshape(1, 1)

    out = pl.pallas_call(
        functools.partial(_mlp_body, nmod, dim),
        grid=(B // _BB,),
        in_specs=[
            pl.BlockSpec((nmod, _BB, 128), lambda i: (0, i, 0)),
            pl.BlockSpec((nmod, _BB, 1), lambda i: (0, i, 0)),
            pl.BlockSpec((_BB, 1), lambda i: (i, 0)),
            pl.BlockSpec((nmod, 128, nhid), lambda i: (0, 0, 0)),
            pl.BlockSpec((1, nhid), lambda i: (0, 0)),
            pl.BlockSpec((1, nhid), lambda i: (0, 0)),
            pl.BlockSpec((1, nhid), lambda i: (0, 0)),
            pl.BlockSpec((1, 1), lambda i: (0, 0)),
        ],
        out_specs=pl.BlockSpec((_BB, 1), lambda i: (i, 0)),
        out_shape=jax.ShapeDtypeStruct((B, 1), jnp.float32),
    )(G43, sel3, t2, w0big, w0t, b0r, W1, b1r)
    return out
